# baseline (speedup 1.0000x reference)
import functools

import jax
import jax.numpy as jnp
from jax import lax
from jax.experimental import pallas as pl
from jax.experimental.pallas import tpu as pltpu

S = 16384
H = 4096
E = 64
K = 8
G = 8
GS = 8
TG = 4
SCALE = 2.5

BS = 256


def _tc_body(h_ref, wt_ref, b_ref, w_ref, i_ref):
    h = h_ref[...]
    wt = wt_ref[...]
    logits = jnp.dot(h, wt, preferred_element_type=jnp.float32)
    scores = 1.0 / (1.0 + jnp.exp(-logits))
    sfc = scores + b_ref[...]

    g3 = sfc.reshape(BS, G, GS)
    m1 = jnp.max(g3, axis=-1)
    io8 = lax.broadcasted_iota(jnp.int32, (BS, G, GS), 2)
    eq = g3 == m1[..., None]
    firstpos = jnp.min(jnp.where(eq, io8, GS), axis=-1, keepdims=True)
    g3m = jnp.where(io8 == firstpos, -jnp.inf, g3)
    m2 = jnp.max(g3m, axis=-1)
    gsum = m1 + m2

    a = gsum[:, :, None]
    bmat = gsum[:, None, :]
    gi = lax.broadcasted_iota(jnp.int32, (BS, G, G), 1)
    hi = lax.broadcasted_iota(jnp.int32, (BS, G, G), 2)
    beats = (bmat > a) | ((bmat == a) & (hi < gi))
    cnt = jnp.sum(beats.astype(jnp.int32), axis=-1)
    sel = (cnt < TG).astype(jnp.float32)
    expmat = (lax.broadcasted_iota(jnp.int32, (G, E), 1) // GS
              == lax.broadcasted_iota(jnp.int32, (G, E), 0)).astype(jnp.float32)
    mask64 = jnp.dot(sel, expmat, preferred_element_type=jnp.float32)
    tmp = jnp.where(mask64 > 0.0, sfc, 0.0)

    io64 = lax.broadcasted_iota(jnp.int32, (BS, E), 1)
    work = tmp
    ws = []
    isel = []
    for _ in range(K):
        mx = jnp.max(work, axis=-1, keepdims=True)
        eqm = work == mx
        idx = jnp.min(jnp.where(eqm, io64, E), axis=-1, keepdims=True)
        hot = io64 == idx
        ws.append(jnp.sum(jnp.where(hot, scores, 0.0), axis=-1, keepdims=True))
        isel.append(idx)
        work = jnp.where(hot, -jnp.inf, work)
    topw = jnp.concatenate(ws, axis=1)
    topi = jnp.concatenate(isel, axis=1)
    denom = jnp.sum(topw, axis=-1, keepdims=True) + 1e-20
    w_ref[...] = topw * (SCALE / denom)
    i_ref[...] = topi


def _make_tc_call():
    return pl.pallas_call(
        _tc_body,
        grid=(S // BS,),
        in_specs=[
            pl.BlockSpec((BS, H), lambda i: (i, 0)),
            pl.BlockSpec((H, E), lambda i: (0, 0)),
            pl.BlockSpec((1, E), lambda i: (0, 0)),
        ],
        out_specs=[
            pl.BlockSpec((BS, K), lambda i: (i, 0)),
            pl.BlockSpec((BS, K), lambda i: (i, 0)),
        ],
        out_shape=[
            jax.ShapeDtypeStruct((S, K), jnp.float32),
            jax.ShapeDtypeStruct((S, K), jnp.int32),
        ],
        compiler_params=pltpu.CompilerParams(
            dimension_semantics=("parallel",)),
    )


@jax.jit
def kernel(hidden_states, weight, e_score_correction_bias):
    wt = weight.T
    b2 = e_score_correction_bias.reshape(1, E)
    topw, topi = _make_tc_call()(hidden_states, wt, b2)
    return topw, topi


# speedup vs baseline: 2.5717x; 2.5717x over previous
import functools

import jax
import jax.numpy as jnp
from jax import lax
from jax.experimental import pallas as pl
from jax.experimental.pallas import tpu as pltpu
from jax.experimental.pallas import tpu_sc as plsc

S = 16384
H = 4096
E = 64
K = 8
G = 8
GS = 8
TG = 4
SCALE = 2.5

BS = 256
NW = 32
TOK = S // NW
NB = TOK // 16

_NEG_INF = float("-inf")


def _mm_body(h_ref, wt_ref, o_ref):
    h = h_ref[...]
    wt = wt_ref[...]
    logits = jnp.dot(h, wt, preferred_element_type=jnp.float32)
    o_ref[...] = 1.0 / (1.0 + jnp.exp(-logits))


def _make_mm_call():
    return pl.pallas_call(
        _mm_body,
        grid=(S // BS,),
        in_specs=[
            pl.BlockSpec((BS, H), lambda i: (i, 0)),
            pl.BlockSpec((H, E), lambda i: (0, 0)),
        ],
        out_specs=pl.BlockSpec((BS, E), lambda i: (i, 0)),
        out_shape=jax.ShapeDtypeStruct((S, E), jnp.float32),
        compiler_params=pltpu.CompilerParams(
            dimension_semantics=("parallel",)),
    )


def _sc_route_body(scores_hbm, bias_hbm, outw_hbm, outi_hbm,
                   sv, sfc, biasv, outw, outi):
    wid = lax.axis_index("s") * 2 + lax.axis_index("c")
    base = wid * TOK
    pltpu.sync_copy(scores_hbm.at[pl.ds(base * E, TOK * E)], sv)
    pltpu.sync_copy(bias_hbm, biasv)
    iota16 = lax.iota(jnp.int32, 16)

    def batch_body(b, _):
        rowidx = b * 16 + iota16
        rowbase = rowidx * E

        gsum = []
        for g in range(G):
            m1 = jnp.full((16,), _NEG_INF, jnp.float32)
            m2 = jnp.full((16,), _NEG_INF, jnp.float32)
            for j in range(GS):
                e = g * GS + j
                p = plsc.load_gather(sv, [rowbase + e])
                x = p + biasv[e, :]
                sfc[e, :] = x
                m2 = jnp.maximum(m2, jnp.minimum(m1, x))
                m1 = jnp.maximum(m1, x)
            gsum.append(m1 + m2)

        sel = []
        for g in range(G):
            cnt = jnp.zeros((16,), jnp.int32)
            for h in range(G):
                if h == g:
                    continue
                if h < g:
                    beat = gsum[h] >= gsum[g]
                else:
                    beat = gsum[h] > gsum[g]
                cnt = cnt + jnp.where(beat, 1, 0)
            sel.append(cnt < TG)

        for g in range(G):
            for j in range(GS):
                e = g * GS + j
                sfc[e, :] = jnp.where(sel[g], sfc[e, :], 0.0)

        vals = [jnp.full((16,), _NEG_INF, jnp.float32) for _ in range(K)]
        idxs = [jnp.zeros((16,), jnp.int32) for _ in range(K)]
        for e in range(E):
            t = sfc[e, :]
            ev = jnp.full((16,), e, jnp.int32)
            c = [t > vals[j] for j in range(K)]
            nv = [jnp.where(c[0], t, vals[0])]
            ni = [jnp.where(c[0], ev, idxs[0])]
            for j in range(1, K):
                shv = jnp.where(c[j - 1], vals[j - 1], t)
                shi = jnp.where(c[j - 1], idxs[j - 1], ev)
                nv.append(jnp.where(c[j], shv, vals[j]))
                ni.append(jnp.where(c[j], shi, idxs[j]))
            vals, idxs = nv, ni

        ps = [plsc.load_gather(sv, [rowbase + idxs[k]]) for k in range(K)]
        denom = ps[0]
        for k in range(1, K):
            denom = denom + ps[k]
        scale = SCALE / (denom + 1e-20)
        outbase = rowidx * K
        for k in range(K):
            plsc.store_scatter(outw, [outbase + k], ps[k] * scale)
            plsc.store_scatter(outi, [outbase + k], idxs[k])
        return 0

    lax.fori_loop(0, NB, batch_body, 0)
    pltpu.sync_copy(outw, outw_hbm.at[pl.ds(base * K, TOK * K)])
    pltpu.sync_copy(outi, outi_hbm.at[pl.ds(base * K, TOK * K)])


def _make_sc_call():
    mesh = plsc.VectorSubcoreMesh(core_axis_name="c", subcore_axis_name="s")
    return functools.partial(
        pl.kernel,
        mesh=mesh,
        out_type=[
            jax.ShapeDtypeStruct((S * K,), jnp.float32),
            jax.ShapeDtypeStruct((S * K,), jnp.int32),
        ],
        scratch_types=[
            pltpu.VMEM((TOK * E,), jnp.float32),
            pltpu.VMEM((E, 16), jnp.float32),
            pltpu.VMEM((E, 16), jnp.float32),
            pltpu.VMEM((TOK * K,), jnp.float32),
            pltpu.VMEM((TOK * K,), jnp.int32),
        ],
        compiler_params=pltpu.CompilerParams(needs_layout_passes=False),
    )(_sc_route_body)


@jax.jit
def kernel(hidden_states, weight, e_score_correction_bias):
    wt = weight.T
    scores = _make_mm_call()(hidden_states, wt)
    biasb = jnp.broadcast_to(e_score_correction_bias[:, None], (E, 16))
    topw, topi = _make_sc_call()(scores.reshape(S * E), biasb)
    return topw.reshape(S, K), topi.reshape(S, K)


# speedup vs baseline: 3.0122x; 1.1713x over previous
import functools

import jax
import jax.numpy as jnp
from jax import lax
from jax.experimental import pallas as pl
from jax.experimental.pallas import tpu as pltpu
from jax.experimental.pallas import tpu_sc as plsc

S = 16384
H = 4096
E = 64
K = 8
G = 8
GS = 8
TG = 4
SCALE = 2.5

BS = 256
NW = 32
TOK = S // NW
NB = TOK // 16

_NEG_INF = float("-inf")


def _mm_body(h_ref, w_ref, o_ref):
    h = h_ref[...]
    w = w_ref[...]
    logits = lax.dot_general(w, h, (((1,), (1,)), ((), ())),
                             preferred_element_type=jnp.float32)
    o_ref[...] = 1.0 / (1.0 + jnp.exp(-logits))


def _make_mm_call():
    return pl.pallas_call(
        _mm_body,
        grid=(S // BS,),
        in_specs=[
            pl.BlockSpec((BS, H), lambda i: (i, 0)),
            pl.BlockSpec((E, H), lambda i: (0, 0)),
        ],
        out_specs=pl.BlockSpec((E, BS), lambda i: (0, i)),
        out_shape=jax.ShapeDtypeStruct((E, S), jnp.float32),
        compiler_params=pltpu.CompilerParams(
            dimension_semantics=("parallel",)),
    )


def _sc_route_body(scores_hbm, bias_hbm, outw_hbm, outi_hbm,
                   svt, pv, sfc, biasv, outw, outi):
    wid = lax.axis_index("s") * 2 + lax.axis_index("c")
    base = wid * TOK
    pltpu.sync_copy(scores_hbm.at[:, pl.ds(base, TOK)], svt)
    pltpu.sync_copy(bias_hbm, biasv)
    iota16 = lax.iota(jnp.int32, 16)

    def batch_body(b, _):
        tok = b * 16

        gsum = []
        for g in range(G):
            m1 = jnp.full((16,), _NEG_INF, jnp.float32)
            m2 = jnp.full((16,), _NEG_INF, jnp.float32)
            for j in range(GS):
                e = g * GS + j
                p = svt[e, pl.ds(tok, 16)]
                pv[pl.ds(e * 16, 16)] = p
                x = p + biasv[pl.ds(e * 16, 16)]
                sfc[pl.ds(e * 16, 16)] = x
                m2 = jnp.maximum(m2, jnp.minimum(m1, x))
                m1 = jnp.maximum(m1, x)
            gsum.append(m1 + m2)

        sel = []
        for g in range(G):
            cnt = jnp.zeros((16,), jnp.int32)
            for h in range(G):
                if h == g:
                    continue
                if h < g:
                    beat = gsum[h] >= gsum[g]
                else:
                    beat = gsum[h] > gsum[g]
                cnt = cnt + jnp.where(beat, 1, 0)
            sel.append(cnt < TG)

        vals = [jnp.full((16,), _NEG_INF, jnp.float32) for _ in range(K)]
        idxs = [jnp.zeros((16,), jnp.int32) for _ in range(K)]
        for e in range(E):
            t = jnp.where(sel[e // GS], sfc[pl.ds(e * 16, 16)], 0.0)
            ev = jnp.full((16,), e, jnp.int32)
            c = [t > vals[j] for j in range(K)]
            nv = [jnp.where(c[0], t, vals[0])]
            ni = [jnp.where(c[0], ev, idxs[0])]
            for j in range(1, K):
                shv = jnp.where(c[j - 1], vals[j - 1], t)
                shi = jnp.where(c[j - 1], idxs[j - 1], ev)
                nv.append(jnp.where(c[j], shv, vals[j]))
                ni.append(jnp.where(c[j], shi, idxs[j]))
            vals, idxs = nv, ni

        ps = [plsc.load_gather(pv, [idxs[k] * 16 + iota16]) for k in range(K)]
        denom = ps[0]
        for k in range(1, K):
            denom = denom + ps[k]
        scale = SCALE / (denom + 1e-20)
        outbase = (tok + iota16) * K
        for k in range(K):
            plsc.store_scatter(outw, [outbase + k], ps[k] * scale)
            plsc.store_scatter(outi, [outbase + k], idxs[k])
        return 0

    lax.fori_loop(0, NB, batch_body, 0)
    pltpu.sync_copy(outw, outw_hbm.at[pl.ds(base * K, TOK * K)])
    pltpu.sync_copy(outi, outi_hbm.at[pl.ds(base * K, TOK * K)])


def _make_sc_call():
    mesh = plsc.VectorSubcoreMesh(core_axis_name="c", subcore_axis_name="s")
    return functools.partial(
        pl.kernel,
        mesh=mesh,
        out_type=[
            jax.ShapeDtypeStruct((S * K,), jnp.float32),
            jax.ShapeDtypeStruct((S * K,), jnp.int32),
        ],
        scratch_types=[
            pltpu.VMEM((E, TOK), jnp.float32),
            pltpu.VMEM((E * 16,), jnp.float32),
            pltpu.VMEM((E * 16,), jnp.float32),
            pltpu.VMEM((E * 16,), jnp.float32),
            pltpu.VMEM((TOK * K,), jnp.float32),
            pltpu.VMEM((TOK * K,), jnp.int32),
        ],
        compiler_params=pltpu.CompilerParams(needs_layout_passes=False),
    )(_sc_route_body)


@jax.jit
def kernel(hidden_states, weight, e_score_correction_bias):
    scores_t = _make_mm_call()(hidden_states, weight)
    biasb = jnp.broadcast_to(
        e_score_correction_bias[:, None], (E, 16)).reshape(E * 16)
    topw, topi = _make_sc_call()(scores_t, biasb)
    return topw.reshape(S, K), topi.reshape(S, K)


# speedup vs baseline: 3.8046x; 1.2631x over previous
import functools

import jax
import jax.numpy as jnp
from jax import lax
from jax.experimental import pallas as pl
from jax.experimental.pallas import tpu as pltpu
from jax.experimental.pallas import tpu_sc as plsc

S = 16384
H = 4096
E = 64
K = 8
G = 8
GS = 8
TG = 4
SCALE = 2.5

CH = 2
SCH = S // CH
BS = 512
NW = 32
TOK = SCH // NW
NB = TOK // 16

_NEG_INF = float("-inf")


def _mm_body(h_ref, w_ref, o_ref):
    h = h_ref[...]
    w = w_ref[...]
    logits = lax.dot_general(w, h, (((1,), (1,)), ((), ())),
                             preferred_element_type=jnp.float32)
    o_ref[...] = 1.0 / (1.0 + jnp.exp(-logits))


def _make_mm_call(chunk):
    nblk = SCH // BS
    return pl.pallas_call(
        _mm_body,
        grid=(nblk,),
        in_specs=[
            pl.BlockSpec((BS, H), lambda i, c=chunk: (c * nblk + i, 0)),
            pl.BlockSpec((E, H), lambda i: (0, 0)),
        ],
        out_specs=pl.BlockSpec((E, BS), lambda i: (0, i)),
        out_shape=jax.ShapeDtypeStruct((E, SCH), jnp.float32),
        compiler_params=pltpu.CompilerParams(
            dimension_semantics=("parallel",)),
    )


def _sc_route_body(scores_hbm, bias_hbm, outw_hbm, outi_hbm,
                   svt, pv, sfc, biasv, outw, outi):
    wid = lax.axis_index("s") * 2 + lax.axis_index("c")
    base = wid * TOK
    pltpu.sync_copy(scores_hbm.at[:, pl.ds(base, TOK)], svt)
    pltpu.sync_copy(bias_hbm, biasv)
    iota16 = lax.iota(jnp.int32, 16)

    def batch_body(b, _):
        tok = b * 16

        gsum = []
        for g in range(G):
            m1 = jnp.full((16,), _NEG_INF, jnp.float32)
            m2 = jnp.full((16,), _NEG_INF, jnp.float32)
            for j in range(GS):
                e = g * GS + j
                p = svt[e, pl.ds(tok, 16)]
                pv[pl.ds(e * 16, 16)] = p
                x = p + biasv[pl.ds(e * 16, 16)]
                sfc[pl.ds(e * 16, 16)] = x
                m2 = jnp.maximum(m2, jnp.minimum(m1, x))
                m1 = jnp.maximum(m1, x)
            gsum.append(m1 + m2)

        sel = []
        for g in range(G):
            cnt = jnp.zeros((16,), jnp.int32)
            for h in range(G):
                if h == g:
                    continue
                if h < g:
                    beat = gsum[h] >= gsum[g]
                else:
                    beat = gsum[h] > gsum[g]
                cnt = cnt + jnp.where(beat, 1, 0)
            sel.append(cnt < TG)

        vals = [jnp.full((16,), _NEG_INF, jnp.float32) for _ in range(K)]
        idxs = [jnp.zeros((16,), jnp.int32) for _ in range(K)]
        for e in range(E):
            t = jnp.where(sel[e // GS], sfc[pl.ds(e * 16, 16)], 0.0)
            ev = jnp.full((16,), e, jnp.int32)
            c = [t > vals[j] for j in range(K)]
            nv = [jnp.where(c[0], t, vals[0])]
            ni = [jnp.where(c[0], ev, idxs[0])]
            for j in range(1, K):
                shv = jnp.where(c[j - 1], vals[j - 1], t)
                shi = jnp.where(c[j - 1], idxs[j - 1], ev)
                nv.append(jnp.where(c[j], shv, vals[j]))
                ni.append(jnp.where(c[j], shi, idxs[j]))
            vals, idxs = nv, ni

        ps = [plsc.load_gather(pv, [idxs[k] * 16 + iota16]) for k in range(K)]
        denom = ps[0]
        for k in range(1, K):
            denom = denom + ps[k]
        scale = SCALE / (denom + 1e-20)
        outbase = (tok + iota16) * K
        for k in range(K):
            plsc.store_scatter(outw, [outbase + k], ps[k] * scale)
            plsc.store_scatter(outi, [outbase + k], idxs[k])
        return 0

    lax.fori_loop(0, NB, batch_body, 0)
    pltpu.sync_copy(outw, outw_hbm.at[pl.ds(base * K, TOK * K)])
    pltpu.sync_copy(outi, outi_hbm.at[pl.ds(base * K, TOK * K)])


def _make_sc_call():
    mesh = plsc.VectorSubcoreMesh(core_axis_name="c", subcore_axis_name="s")
    return functools.partial(
        pl.kernel,
        mesh=mesh,
        out_type=[
            jax.ShapeDtypeStruct((SCH * K,), jnp.float32),
            jax.ShapeDtypeStruct((SCH * K,), jnp.int32),
        ],
        scratch_types=[
            pltpu.VMEM((E, TOK), jnp.float32),
            pltpu.VMEM((E * 16,), jnp.float32),
            pltpu.VMEM((E * 16,), jnp.float32),
            pltpu.VMEM((E * 16,), jnp.float32),
            pltpu.VMEM((TOK * K,), jnp.float32),
            pltpu.VMEM((TOK * K,), jnp.int32),
        ],
        compiler_params=pltpu.CompilerParams(needs_layout_passes=False),
    )(_sc_route_body)


@jax.jit
def kernel(hidden_states, weight, e_score_correction_bias):
    biasb = jnp.broadcast_to(
        e_score_correction_bias[:, None], (E, 16)).reshape(E * 16)
    sc_call = _make_sc_call()
    tws, tis = [], []
    for c in range(CH):
        scores_t = _make_mm_call(c)(hidden_states, weight)
        topw, topi = sc_call(scores_t, biasb)
        tws.append(topw.reshape(SCH, K))
        tis.append(topi.reshape(SCH, K))
    return (jnp.concatenate(tws, axis=0), jnp.concatenate(tis, axis=0))
